# baseline (speedup 1.0000x reference)
import functools

import jax
import jax.numpy as jnp
from jax.experimental import pallas as pl

B, T, D, E, V = 128, 32, 32, 256, 32
MASK_ID = 0


def _tc_body(ids_ref, pp_ref, et_ref, gt_ref, w_ref, b_ref, out_ref, valid_ref):
    i = pl.program_id(0)
    ids = ids_ref[i, :]
    onehot = (ids[:, None] == jax.lax.broadcasted_iota(jnp.int32, (D, V), 1)
              ).astype(jnp.float32)
    gamma = jnp.dot(onehot, gt_ref[...], preferred_element_type=jnp.float32)
    feat = jnp.dot(onehot, et_ref[...], preferred_element_type=jnp.float32)
    w = w_ref[0, :]
    bias = b_ref[0, :]
    a = gamma * w[None, :]
    c = gamma * bias[None, :] + feat
    pp = pp_ref[0]
    out_ref[0] = pp[:, :, None] * a[None] + c[None]
    valid_ref[0] = jnp.broadcast_to((ids != MASK_ID)[None, :], (T, D))


def kernel(proprio, dof_ids, embed_table, gamma_table, W, b):
    dof_ids = dof_ids.astype(jnp.int32)
    b2 = b.reshape(1, E)
    grid = (B,)
    tokens, valid = pl.pallas_call(
        _tc_body,
        grid=grid,
        in_specs=[
            pl.BlockSpec((B, D), lambda i: (0, 0)),
            pl.BlockSpec((1, T, D), lambda i: (i, 0, 0)),
            pl.BlockSpec((V, E), lambda i: (0, 0)),
            pl.BlockSpec((V, E), lambda i: (0, 0)),
            pl.BlockSpec((1, E), lambda i: (0, 0)),
            pl.BlockSpec((1, E), lambda i: (0, 0)),
        ],
        out_specs=[
            pl.BlockSpec((1, T, D, E), lambda i: (i, 0, 0, 0)),
            pl.BlockSpec((1, T, D), lambda i: (i, 0, 0)),
        ],
        out_shape=[
            jax.ShapeDtypeStruct((B, T, D, E), jnp.float32),
            jax.ShapeDtypeStruct((B, T, D), jnp.bool_),
        ],
    )(dof_ids, proprio, embed_table, gamma_table, W, b2)
    return tokens, valid


# speedup vs baseline: 1.7853x; 1.7853x over previous
import functools

import jax
import jax.numpy as jnp
from jax.experimental import pallas as pl

B, T, D, E, V = 128, 32, 32, 256, 32
MASK_ID = 0


G = 4


def _tc_body(ids_ref, pp_ref, et_ref, gt_ref, w_ref, b_ref, out_ref, valid_ref):
    i = pl.program_id(0)
    ids = ids_ref[pl.ds(i * G, G), :]
    w = w_ref[0, :]
    bias = b_ref[0, :]
    for g in range(G):
        idsg = ids[g, :]
        onehot = (idsg[:, None] == jax.lax.broadcasted_iota(jnp.int32, (D, V), 1)
                  ).astype(jnp.float32)
        gamma = jnp.dot(onehot, gt_ref[...], preferred_element_type=jnp.float32)
        feat = jnp.dot(onehot, et_ref[...], preferred_element_type=jnp.float32)
        a = gamma * w[None, :]
        c = gamma * bias[None, :] + feat
        pp = pp_ref[g]
        out_ref[g] = pp[:, :, None] * a[None] + c[None]
    valid_ref[...] = jnp.broadcast_to((ids != MASK_ID)[:, None, :], (G, T, D))


def kernel(proprio, dof_ids, embed_table, gamma_table, W, b):
    dof_ids = dof_ids.astype(jnp.int32)
    b2 = b.reshape(1, E)
    grid = (B // G,)
    tokens, valid = pl.pallas_call(
        _tc_body,
        grid=grid,
        in_specs=[
            pl.BlockSpec((B, D), lambda i: (0, 0)),
            pl.BlockSpec((G, T, D), lambda i: (i, 0, 0)),
            pl.BlockSpec((V, E), lambda i: (0, 0)),
            pl.BlockSpec((V, E), lambda i: (0, 0)),
            pl.BlockSpec((1, E), lambda i: (0, 0)),
            pl.BlockSpec((1, E), lambda i: (0, 0)),
        ],
        out_specs=[
            pl.BlockSpec((G, T, D, E), lambda i: (i, 0, 0, 0)),
            pl.BlockSpec((G, T, D), lambda i: (i, 0, 0)),
        ],
        out_shape=[
            jax.ShapeDtypeStruct((B, T, D, E), jnp.float32),
            jax.ShapeDtypeStruct((B, T, D), jnp.bool_),
        ],
    )(dof_ids, proprio, embed_table, gamma_table, W, b2)
    return tokens, valid


# speedup vs baseline: 1.9478x; 1.0910x over previous
import functools

import jax
import jax.numpy as jnp
from jax.experimental import pallas as pl

B, T, D, E, V = 128, 32, 32, 256, 32
MASK_ID = 0


G = 8


def _tc_body(ids_ref, pp_ref, et_ref, gt_ref, w_ref, b_ref, out_ref, valid_ref):
    i = pl.program_id(0)
    ids = ids_ref[pl.ds(i * G, G), :]
    w = w_ref[0, :]
    bias = b_ref[0, :]
    for g in range(G):
        idsg = ids[g, :]
        onehot = (idsg[:, None] == jax.lax.broadcasted_iota(jnp.int32, (D, V), 1)
                  ).astype(jnp.float32)
        gamma = jnp.dot(onehot, gt_ref[...], preferred_element_type=jnp.float32)
        feat = jnp.dot(onehot, et_ref[...], preferred_element_type=jnp.float32)
        a = gamma * w[None, :]
        c = gamma * bias[None, :] + feat
        pp = pp_ref[g]
        out_ref[g] = pp[:, :, None] * a[None] + c[None]
    valid_ref[...] = jnp.broadcast_to((ids != MASK_ID)[:, None, :], (G, T, D))


def kernel(proprio, dof_ids, embed_table, gamma_table, W, b):
    dof_ids = dof_ids.astype(jnp.int32)
    b2 = b.reshape(1, E)
    grid = (B // G,)
    tokens, valid = pl.pallas_call(
        _tc_body,
        grid=grid,
        in_specs=[
            pl.BlockSpec((B, D), lambda i: (0, 0)),
            pl.BlockSpec((G, T, D), lambda i: (i, 0, 0)),
            pl.BlockSpec((V, E), lambda i: (0, 0)),
            pl.BlockSpec((V, E), lambda i: (0, 0)),
            pl.BlockSpec((1, E), lambda i: (0, 0)),
            pl.BlockSpec((1, E), lambda i: (0, 0)),
        ],
        out_specs=[
            pl.BlockSpec((G, T, D, E), lambda i: (i, 0, 0, 0)),
            pl.BlockSpec((G, T, D), lambda i: (i, 0, 0)),
        ],
        out_shape=[
            jax.ShapeDtypeStruct((B, T, D, E), jnp.float32),
            jax.ShapeDtypeStruct((B, T, D), jnp.bool_),
        ],
    )(dof_ids, proprio, embed_table, gamma_table, W, b2)
    return tokens, valid
